# Initial kernel scaffold; baseline (speedup 1.0000x reference)
import jax
import jax.numpy as jnp
from jax.experimental import pallas as pl


def kernel(x, W_bias, W_factor):
    raise NotImplementedError("write your pallas kernel here")


# speedup vs baseline: 13.6891x; 13.6891x over previous
import functools

import jax
import jax.numpy as jnp
from jax import lax
from jax.experimental import pallas as pl
from jax.experimental.pallas import tpu as pltpu
from jax.experimental.pallas import tpu_sc as plsc

F_NUM = 26
L_NUM = 16
BATCH = 4096
D = F_NUM * L_NUM
NW = 32
EPW = BATCH // NW
NB = 4
RPB = NB * F_NUM
NBLK = EPW // NB
GRP = 4
NGRP = NBLK // GRP


def _fm_body(x_hbm, wf_hbm, wb_hbm, out_hbm,
             idx_v, rows_v, bias_v, out_v, sf0, sf1, sb0, sb1):
    cid = lax.axis_index("c")
    sid = lax.axis_index("s")
    wid = sid * 2 + cid

    pltpu.sync_copy(x_hbm.at[pl.ds(wid * NBLK, NBLK)], idx_v)

    sf = (sf0, sf1)
    sb = (sb0, sb1)

    def start(blk, buf):
        pltpu.make_async_copy(
            wf_hbm.at[idx_v.at[blk]], rows_v.at[buf], sf[buf]).start()
        pltpu.make_async_copy(
            wb_hbm.at[idx_v.at[blk]], bias_v.at[buf, pl.ds(0, RPB)],
            sb[buf]).start()

    def wait(blk, buf):
        pltpu.make_async_copy(
            wf_hbm.at[idx_v.at[blk]], rows_v.at[buf], sf[buf]).wait()
        pltpu.make_async_copy(
            wb_hbm.at[idx_v.at[blk]], bias_v.at[buf, pl.ds(0, RPB)],
            sb[buf]).wait()

    start(0, 0)
    start(1, 1)

    lane = lax.iota(jnp.int32, 16)
    tail_mask = lane < (F_NUM - 16)
    zeros16 = jnp.zeros((16,), jnp.float32)
    perms = [lane ^ (1 << k) for k in range(4)]

    def lane_sum(v):
        for p in perms:
            v = v + v.at[p].get(mode="promise_in_bounds", unique_indices=True)
        return v

    def group_body(g, _):
        out_vec = zeros16
        for t in range(GRP):
            blk = GRP * g + t
            buf = t % 2
            wait(blk, buf)
            for e in range(NB):
                base = e * F_NUM
                b0 = bias_v[buf, pl.ds(base, 16)]
                b1 = bias_v[buf, pl.ds(base + 16, 16)]
                acc = b0 + jnp.where(tail_mask, b1, zeros16)

                def outer(i, acc):
                    def inner(j, acc):
                        v1 = rows_v[buf, base + i, pl.ds(j * 16, 16)]
                        v2 = rows_v[buf, base + j, pl.ds(i * 16, 16)]
                        return acc + v1 * v2
                    return lax.fori_loop(i + 1, F_NUM, inner, acc)

                acc = lax.fori_loop(0, F_NUM, outer, acc)
                r = lane_sum(acc)
                out_vec = jnp.where(lane == (GRP * t + e), r, out_vec)
            @pl.when(blk + 2 < NBLK)
            def _():
                start(blk + 2, buf)
        out_v[pl.ds(g * 16, 16)] = out_vec
        return 0

    lax.fori_loop(0, NGRP, group_body, 0)
    pltpu.sync_copy(out_v, out_hbm.at[pl.ds(wid * EPW, EPW)])


@jax.jit
def kernel(x, W_bias, W_factor):
    x2 = x.astype(jnp.int32).reshape(NW * NBLK, RPB)
    wf2 = W_factor.reshape(-1, D)
    wb1 = W_bias.reshape(-1)

    fm = pl.kernel(
        _fm_body,
        out_type=jax.ShapeDtypeStruct((BATCH,), jnp.float32),
        mesh=plsc.VectorSubcoreMesh(core_axis_name="c", subcore_axis_name="s"),
        compiler_params=pltpu.CompilerParams(use_tc_tiling_on_sc=False),
        scratch_types=[
            pltpu.VMEM((NBLK, RPB), jnp.int32),
            pltpu.VMEM((2, RPB, D), jnp.float32),
            pltpu.VMEM((2, RPB + 8), jnp.float32),
            pltpu.VMEM((EPW,), jnp.float32),
            pltpu.SemaphoreType.DMA,
            pltpu.SemaphoreType.DMA,
            pltpu.SemaphoreType.DMA,
            pltpu.SemaphoreType.DMA,
        ],
    )
    out = fm(x2, wf2, wb1)
    return out.reshape(BATCH, 1)


# speedup vs baseline: 14.4874x; 1.0583x over previous
import functools

import jax
import jax.numpy as jnp
from jax import lax
from jax.experimental import pallas as pl
from jax.experimental.pallas import tpu as pltpu
from jax.experimental.pallas import tpu_sc as plsc

F_NUM = 26
L_NUM = 16
BATCH = 4096
D = F_NUM * L_NUM
DP = 512
NW = 32
EPW = BATCH // NW
NB = 4
RPB = NB * F_NUM
NBLK = EPW // NB
IPW = EPW * F_NUM
GRP = 4
NGRP = NBLK // GRP


def _fm_body(x_hbm, wf_hbm, out_hbm, idx_v, rows_v, out_v, sf0, sf1):
    cid = lax.axis_index("c")
    sid = lax.axis_index("s")
    wid = sid * 2 + cid

    pltpu.sync_copy(x_hbm.at[pl.ds(wid * IPW, IPW)], idx_v)

    sf = (sf0, sf1)

    def start(blk, buf):
        pltpu.make_async_copy(
            wf_hbm.at[idx_v.at[pl.ds(blk * RPB, RPB)]], rows_v.at[buf],
            sf[buf]).start()

    def wait(blk, buf):
        pltpu.make_async_copy(
            wf_hbm.at[idx_v.at[pl.ds(blk * RPB, RPB)]], rows_v.at[buf],
            sf[buf]).wait()

    start(0, 0)
    start(1, 1)

    lane = lax.iota(jnp.int32, 16)
    zeros16 = jnp.zeros((16,), jnp.float32)
    perms = [lane ^ (1 << k) for k in range(4)]

    def lane_sum(v):
        for p in perms:
            v = v + v.at[p].get(mode="promise_in_bounds", unique_indices=True)
        return v

    def group_body(g, _):
        out_vec = zeros16
        for t in range(GRP):
            blk = GRP * g + t
            buf = t % 2
            wait(blk, buf)
            for e in range(NB):
                base = e * F_NUM

                def bias_add(i, acc):
                    return acc + rows_v[buf, base + i, pl.ds(D, 16)]

                acc = lax.fori_loop(0, F_NUM, bias_add, zeros16)

                def outer(i, acc):
                    def inner(j, acc):
                        v1 = rows_v[buf, base + i, pl.ds(j * 16, 16)]
                        v2 = rows_v[buf, base + j, pl.ds(i * 16, 16)]
                        return acc + v1 * v2
                    return lax.fori_loop(i + 1, F_NUM, inner, acc)

                acc = lax.fori_loop(0, F_NUM, outer, acc)
                r = lane_sum(acc)
                out_vec = jnp.where(lane == (GRP * t + e), r, out_vec)
            @pl.when(blk + 2 < NBLK)
            def _():
                start(blk + 2, buf)
        out_v[pl.ds(g * 16, 16)] = out_vec
        return 0

    lax.fori_loop(0, NGRP, group_body, 0)
    pltpu.sync_copy(out_v, out_hbm.at[pl.ds(wid * EPW, EPW)])


@jax.jit
def kernel(x, W_bias, W_factor):
    x1 = x.astype(jnp.int32).reshape(-1)
    wfp = jnp.concatenate(
        [W_factor.reshape(-1, D), W_bias.reshape(-1, 1),
         jnp.zeros((W_bias.shape[0], DP - D - 1), jnp.float32)], axis=1)

    fm = pl.kernel(
        _fm_body,
        out_type=jax.ShapeDtypeStruct((BATCH,), jnp.float32),
        mesh=plsc.VectorSubcoreMesh(core_axis_name="c", subcore_axis_name="s"),
        scratch_types=[
            pltpu.VMEM((IPW,), jnp.int32),
            pltpu.VMEM((2, RPB, DP), jnp.float32),
            pltpu.VMEM((EPW,), jnp.float32),
            pltpu.SemaphoreType.DMA,
            pltpu.SemaphoreType.DMA,
        ],
    )
    out = fm(x1, wfp)
    return out.reshape(BATCH, 1)


# speedup vs baseline: 35.1405x; 2.4256x over previous
import functools

import jax
import jax.numpy as jnp
from jax import lax
from jax.experimental import pallas as pl
from jax.experimental.pallas import tpu as pltpu
from jax.experimental.pallas import tpu_sc as plsc

F_NUM = 26
L_NUM = 16
BATCH = 4096
V_NUM = 100000
D = F_NUM * L_NUM
DP = 512
NW = 32
EPW = BATCH // NW
NB = 4
RPB = NB * F_NUM
NBLK = EPW // NB
IPW = EPW * F_NUM
GRP = 4
NGRP = NBLK // GRP

BV = 512
NVB = (V_NUM + BV - 1) // BV


def _tp_body(wt_ref, wb_ref, out_ref):
    t = wt_ref[...].T
    out_ref[:, 0:D] = t
    out_ref[:, D:D + 1] = wb_ref[...]
    out_ref[:, D + 1:DP] = jnp.zeros((BV, DP - D - 1), jnp.float32)


def _build_table(wt, wb):
    return pl.pallas_call(
        _tp_body,
        grid=(NVB,),
        in_specs=[
            pl.BlockSpec((D, BV), lambda i: (0, i)),
            pl.BlockSpec((BV, 1), lambda i: (i, 0)),
        ],
        out_specs=pl.BlockSpec((BV, DP), lambda i: (i, 0)),
        out_shape=jax.ShapeDtypeStruct((NVB * BV, DP), jnp.float32),
    )(wt, wb)


def _fm_body(x_hbm, wf_hbm, out_hbm, idx_v, rows_v, out_v, sf0, sf1):
    cid = lax.axis_index("c")
    sid = lax.axis_index("s")
    wid = sid * 2 + cid

    pltpu.sync_copy(x_hbm.at[pl.ds(wid * IPW, IPW)], idx_v)

    sf = (sf0, sf1)

    def start(blk, buf):
        pltpu.make_async_copy(
            wf_hbm.at[idx_v.at[pl.ds(blk * RPB, RPB)]], rows_v.at[buf],
            sf[buf]).start()

    def wait(blk, buf):
        pltpu.make_async_copy(
            wf_hbm.at[idx_v.at[pl.ds(blk * RPB, RPB)]], rows_v.at[buf],
            sf[buf]).wait()

    start(0, 0)
    start(1, 1)

    lane = lax.iota(jnp.int32, 16)
    zeros16 = jnp.zeros((16,), jnp.float32)
    perms = [lane ^ (1 << k) for k in range(4)]

    def lane_sum(v):
        for p in perms:
            v = v + v.at[p].get(mode="promise_in_bounds", unique_indices=True)
        return v

    def group_body(g, _):
        out_vec = zeros16
        for t in range(GRP):
            blk = GRP * g + t
            buf = t % 2
            wait(blk, buf)
            for e in range(NB):
                base = e * F_NUM

                def bias_add(i, acc):
                    return acc + rows_v[buf, base + i, pl.ds(D, 16)]

                acc = lax.fori_loop(0, F_NUM, bias_add, zeros16)

                def outer(i, acc):
                    def inner(j, acc):
                        v1 = rows_v[buf, base + i, pl.ds(j * 16, 16)]
                        v2 = rows_v[buf, base + j, pl.ds(i * 16, 16)]
                        return acc + v1 * v2
                    return lax.fori_loop(i + 1, F_NUM, inner, acc)

                acc = lax.fori_loop(0, F_NUM, outer, acc)
                r = lane_sum(acc)
                out_vec = jnp.where(lane == (GRP * t + e), r, out_vec)
            @pl.when(blk + 2 < NBLK)
            def _():
                start(blk + 2, buf)
        out_v[pl.ds(g * 16, 16)] = out_vec
        return 0

    lax.fori_loop(0, NGRP, group_body, 0)
    pltpu.sync_copy(out_v, out_hbm.at[pl.ds(wid * EPW, EPW)])


@jax.jit
def kernel(x, W_bias, W_factor):
    x1 = x.astype(jnp.int32).reshape(-1)
    wt = W_factor.reshape(V_NUM, D).T
    wfp = _build_table(wt, W_bias)

    fm = pl.kernel(
        _fm_body,
        out_type=jax.ShapeDtypeStruct((BATCH,), jnp.float32),
        mesh=plsc.VectorSubcoreMesh(core_axis_name="c", subcore_axis_name="s"),
        scratch_types=[
            pltpu.VMEM((IPW,), jnp.int32),
            pltpu.VMEM((2, RPB, DP), jnp.float32),
            pltpu.VMEM((EPW,), jnp.float32),
            pltpu.SemaphoreType.DMA,
            pltpu.SemaphoreType.DMA,
        ],
    )
    out = fm(x1, wfp)
    return out.reshape(BATCH, 1)


# speedup vs baseline: 49.3666x; 1.4048x over previous
import functools

import jax
import jax.numpy as jnp
from jax import lax
from jax.experimental import pallas as pl
from jax.experimental.pallas import tpu as pltpu
from jax.experimental.pallas import tpu_sc as plsc

F_NUM = 26
L_NUM = 16
BATCH = 4096
V_NUM = 100000
D = F_NUM * L_NUM
DP = 512
NW = 32
EPW = BATCH // NW
NB = 4
RPB = NB * F_NUM
NBLK = EPW // NB
IPW = EPW * F_NUM
GRP = 4
NGRP = NBLK // GRP

BV = 512
NVB = (V_NUM + BV - 1) // BV


def _tp_body(wt_ref, wb_ref, out_ref):
    t = wt_ref[...].T
    out_ref[:, 0:D] = t
    out_ref[:, D:D + 1] = wb_ref[...]
    out_ref[:, D + 1:DP] = jnp.zeros((BV, DP - D - 1), jnp.float32)


def _build_table(wt, wb):
    return pl.pallas_call(
        _tp_body,
        grid=(NVB,),
        in_specs=[
            pl.BlockSpec((D, BV), lambda i: (0, i)),
            pl.BlockSpec((BV, 1), lambda i: (i, 0)),
        ],
        out_specs=pl.BlockSpec((BV, DP), lambda i: (i, 0)),
        out_shape=jax.ShapeDtypeStruct((NVB * BV, DP), jnp.float32),
    )(wt, wb)


def _fm_body(x_hbm, wf_hbm, out_hbm, idx_v, rows_v, out_v, sf0, sf1):
    cid = lax.axis_index("c")
    sid = lax.axis_index("s")
    wid = sid * 2 + cid

    pltpu.sync_copy(x_hbm.at[pl.ds(wid * IPW, IPW)], idx_v)

    sf = (sf0, sf1)

    def start(blk, buf):
        pltpu.make_async_copy(
            wf_hbm.at[idx_v.at[pl.ds(blk * RPB, RPB)]], rows_v.at[buf],
            sf[buf]).start()

    def wait(blk, buf):
        pltpu.make_async_copy(
            wf_hbm.at[idx_v.at[pl.ds(blk * RPB, RPB)]], rows_v.at[buf],
            sf[buf]).wait()

    start(0, 0)
    start(1, 1)

    lane = lax.iota(jnp.int32, 16)
    zeros16 = jnp.zeros((16,), jnp.float32)
    perms = [lane ^ (1 << k) for k in range(4)]

    def lane_sum(v):
        for p in perms:
            v = v + v.at[p].get(mode="promise_in_bounds", unique_indices=True)
        return v

    def group_body(g, _):
        out_vec = zeros16
        for t in range(GRP):
            blk = GRP * g + t
            buf = t % 2
            wait(blk, buf)

            def elem_body(e, out_vec):
                base = e * F_NUM
                acc = rows_v[buf, base, pl.ds(D, 16)]
                for i in range(1, F_NUM):
                    acc = acc + rows_v[buf, base + i, pl.ds(D, 16)]
                for i in range(F_NUM):
                    for j in range(i + 1, F_NUM):
                        v1 = rows_v[buf, base + i, pl.ds(j * 16, 16)]
                        v2 = rows_v[buf, base + j, pl.ds(i * 16, 16)]
                        acc = acc + v1 * v2
                r = lane_sum(acc)
                return jnp.where(lane == (GRP * t + e), r, out_vec)

            out_vec = lax.fori_loop(0, NB, elem_body, out_vec)
            @pl.when(blk + 2 < NBLK)
            def _():
                start(blk + 2, buf)
        out_v[pl.ds(g * 16, 16)] = out_vec
        return 0

    lax.fori_loop(0, NGRP, group_body, 0)
    pltpu.sync_copy(out_v, out_hbm.at[pl.ds(wid * EPW, EPW)])


@jax.jit
def kernel(x, W_bias, W_factor):
    x1 = x.astype(jnp.int32).reshape(-1)
    wt = W_factor.reshape(V_NUM, D).T
    wfp = _build_table(wt, W_bias)

    fm = pl.kernel(
        _fm_body,
        out_type=jax.ShapeDtypeStruct((BATCH,), jnp.float32),
        mesh=plsc.VectorSubcoreMesh(core_axis_name="c", subcore_axis_name="s"),
        scratch_types=[
            pltpu.VMEM((IPW,), jnp.int32),
            pltpu.VMEM((2, RPB, DP), jnp.float32),
            pltpu.VMEM((EPW,), jnp.float32),
            pltpu.SemaphoreType.DMA,
            pltpu.SemaphoreType.DMA,
        ],
    )
    out = fm(x1, wfp)
    return out.reshape(BATCH, 1)


# speedup vs baseline: 58.8886x; 1.1929x over previous
import functools

import jax
import jax.numpy as jnp
from jax import lax
from jax.experimental import pallas as pl
from jax.experimental.pallas import tpu as pltpu
from jax.experimental.pallas import tpu_sc as plsc

F_NUM = 26
L_NUM = 16
BATCH = 4096
V_NUM = 100000
D = F_NUM * L_NUM
DP = 512
NW = 32
EPW = BATCH // NW
NB = 4
RPB = NB * F_NUM
NBLK = EPW // NB
IPW = EPW * F_NUM
GRP = 4
NGRP = NBLK // GRP

BV = 1024
NVB = (V_NUM + BV - 1) // BV


def _tp_body(wt_ref, wb_ref, out_ref):
    t = wt_ref[...].T
    out_ref[:, 0:D] = t
    out_ref[:, D:D + 1] = wb_ref[...]
    out_ref[:, D + 1:DP] = jnp.zeros((BV, DP - D - 1), jnp.float32)


def _build_table(wt, wb):
    return pl.pallas_call(
        _tp_body,
        grid=(NVB,),
        in_specs=[
            pl.BlockSpec((D, BV), lambda i: (0, i)),
            pl.BlockSpec((BV, 1), lambda i: (i, 0)),
        ],
        out_specs=pl.BlockSpec((BV, DP), lambda i: (i, 0)),
        out_shape=jax.ShapeDtypeStruct((NVB * BV, DP), jnp.float32),
    )(wt, wb)


def _fm_body(x_hbm, wf_hbm, out_hbm, idx_v, rows_v, out_v, sf0, sf1):
    cid = lax.axis_index("c")
    sid = lax.axis_index("s")
    wid = sid * 2 + cid

    pltpu.sync_copy(x_hbm.at[pl.ds(wid * IPW, IPW)], idx_v)

    sf = (sf0, sf1)

    def start(blk, buf):
        pltpu.make_async_copy(
            wf_hbm.at[idx_v.at[pl.ds(blk * RPB, RPB)]], rows_v.at[buf],
            sf[buf]).start()

    def wait(blk, buf):
        pltpu.make_async_copy(
            wf_hbm.at[idx_v.at[pl.ds(blk * RPB, RPB)]], rows_v.at[buf],
            sf[buf]).wait()

    start(0, 0)
    start(1, 1)

    lane = lax.iota(jnp.int32, 16)
    zeros16 = jnp.zeros((16,), jnp.float32)
    perms = [lane ^ (1 << k) for k in range(4)]

    def lane_sum(v):
        for p in perms:
            v = v + v.at[p].get(mode="promise_in_bounds", unique_indices=True)
        return v

    def group_body(g, _):
        out_vec = zeros16
        for t in range(GRP):
            blk = GRP * g + t
            buf = t % 2
            wait(blk, buf)

            def elem_body(e, out_vec):
                base = e * F_NUM
                acc = rows_v[buf, base, pl.ds(D, 16)]
                for i in range(1, F_NUM):
                    acc = acc + rows_v[buf, base + i, pl.ds(D, 16)]
                for i in range(F_NUM):
                    for j in range(i + 1, F_NUM):
                        v1 = rows_v[buf, base + i, pl.ds(j * 16, 16)]
                        v2 = rows_v[buf, base + j, pl.ds(i * 16, 16)]
                        acc = acc + v1 * v2
                r = lane_sum(acc)
                return jnp.where(lane == (GRP * t + e), r, out_vec)

            out_vec = lax.fori_loop(0, NB, elem_body, out_vec)
            @pl.when(blk + 2 < NBLK)
            def _():
                start(blk + 2, buf)
        out_v[pl.ds(g * 16, 16)] = out_vec
        return 0

    lax.fori_loop(0, NGRP, group_body, 0)
    pltpu.sync_copy(out_v, out_hbm.at[pl.ds(wid * EPW, EPW)])


@jax.jit
def kernel(x, W_bias, W_factor):
    x1 = x.astype(jnp.int32).reshape(-1)
    wt = W_factor.reshape(V_NUM, D).T
    wfp = _build_table(wt, W_bias)

    fm = pl.kernel(
        _fm_body,
        out_type=jax.ShapeDtypeStruct((BATCH,), jnp.float32),
        mesh=plsc.VectorSubcoreMesh(core_axis_name="c", subcore_axis_name="s"),
        scratch_types=[
            pltpu.VMEM((IPW,), jnp.int32),
            pltpu.VMEM((2, RPB, DP), jnp.float32),
            pltpu.VMEM((EPW,), jnp.float32),
            pltpu.SemaphoreType.DMA,
            pltpu.SemaphoreType.DMA,
        ],
    )
    out = fm(x1, wfp)
    return out.reshape(BATCH, 1)


# speedup vs baseline: 64.2715x; 1.0914x over previous
import functools

import jax
import jax.numpy as jnp
from jax import lax
from jax.experimental import pallas as pl
from jax.experimental.pallas import tpu as pltpu
from jax.experimental.pallas import tpu_sc as plsc

F_NUM = 26
L_NUM = 16
BATCH = 4096
V_NUM = 100000
D = F_NUM * L_NUM
DP = 512
NW = 32
EPW = BATCH // NW
NB = 4
RPB = NB * F_NUM
NBLK = EPW // NB
IPW = EPW * F_NUM
GRP = 4
NGRP = NBLK // GRP

BV = 2048
NVB = (V_NUM + BV - 1) // BV


def _tp_body(wt_ref, wb_ref, out_ref):
    t = wt_ref[...].T
    out_ref[:, 0:D] = t
    out_ref[:, D:D + 1] = wb_ref[...]
    out_ref[:, D + 1:DP] = jnp.zeros((BV, DP - D - 1), jnp.float32)


def _build_table(wt, wb):
    return pl.pallas_call(
        _tp_body,
        grid=(NVB,),
        in_specs=[
            pl.BlockSpec((D, BV), lambda i: (0, i)),
            pl.BlockSpec((BV, 1), lambda i: (i, 0)),
        ],
        out_specs=pl.BlockSpec((BV, DP), lambda i: (i, 0)),
        out_shape=jax.ShapeDtypeStruct((NVB * BV, DP), jnp.float32),
    )(wt, wb)


def _fm_body(x_hbm, wf_hbm, out_hbm, idx_v, rows_v, out_v, sf0, sf1):
    cid = lax.axis_index("c")
    sid = lax.axis_index("s")
    wid = sid * 2 + cid

    pltpu.sync_copy(x_hbm.at[pl.ds(wid * IPW, IPW)], idx_v)

    sf = (sf0, sf1)

    def start(blk, buf):
        pltpu.make_async_copy(
            wf_hbm.at[idx_v.at[pl.ds(blk * RPB, RPB)]], rows_v.at[buf],
            sf[buf]).start()

    def wait(blk, buf):
        pltpu.make_async_copy(
            wf_hbm.at[idx_v.at[pl.ds(blk * RPB, RPB)]], rows_v.at[buf],
            sf[buf]).wait()

    start(0, 0)
    start(1, 1)

    lane = lax.iota(jnp.int32, 16)
    zeros16 = jnp.zeros((16,), jnp.float32)
    perms = [lane ^ (1 << k) for k in range(4)]

    def lane_sum(v):
        for p in perms:
            v = v + v.at[p].get(mode="promise_in_bounds", unique_indices=True)
        return v

    def group_body(g, _):
        out_vec = zeros16
        for t in range(GRP):
            blk = GRP * g + t
            buf = t % 2
            wait(blk, buf)

            def elem_body(e, out_vec):
                base = e * F_NUM
                acc = rows_v[buf, base, pl.ds(D, 16)]
                for i in range(1, F_NUM):
                    acc = acc + rows_v[buf, base + i, pl.ds(D, 16)]
                for i in range(F_NUM):
                    for j in range(i + 1, F_NUM):
                        v1 = rows_v[buf, base + i, pl.ds(j * 16, 16)]
                        v2 = rows_v[buf, base + j, pl.ds(i * 16, 16)]
                        acc = acc + v1 * v2
                r = lane_sum(acc)
                return jnp.where(lane == (GRP * t + e), r, out_vec)

            out_vec = lax.fori_loop(0, NB, elem_body, out_vec)
            @pl.when(blk + 2 < NBLK)
            def _():
                start(blk + 2, buf)
        out_v[pl.ds(g * 16, 16)] = out_vec
        return 0

    lax.fori_loop(0, NGRP, group_body, 0)
    pltpu.sync_copy(out_v, out_hbm.at[pl.ds(wid * EPW, EPW)])


@jax.jit
def kernel(x, W_bias, W_factor):
    x1 = x.astype(jnp.int32).reshape(-1)
    wt = W_factor.reshape(V_NUM, D).T
    wfp = _build_table(wt, W_bias)

    fm = pl.kernel(
        _fm_body,
        out_type=jax.ShapeDtypeStruct((BATCH,), jnp.float32),
        mesh=plsc.VectorSubcoreMesh(core_axis_name="c", subcore_axis_name="s"),
        scratch_types=[
            pltpu.VMEM((IPW,), jnp.int32),
            pltpu.VMEM((2, RPB, DP), jnp.float32),
            pltpu.VMEM((EPW,), jnp.float32),
            pltpu.SemaphoreType.DMA,
            pltpu.SemaphoreType.DMA,
        ],
    )
    out = fm(x1, wfp)
    return out.reshape(BATCH, 1)


# speedup vs baseline: 65.2401x; 1.0151x over previous
import functools

import jax
import jax.numpy as jnp
from jax import lax
from jax.experimental import pallas as pl
from jax.experimental.pallas import tpu as pltpu
from jax.experimental.pallas import tpu_sc as plsc

F_NUM = 26
L_NUM = 16
BATCH = 4096
V_NUM = 100000
D = F_NUM * L_NUM
DP = 512
NW = 32
EPW = BATCH // NW
NB = 4
RPB = NB * F_NUM
NBLK = EPW // NB
IPW = EPW * F_NUM
GRP = 4
NGRP = NBLK // GRP

BV = 4096
NVB = (V_NUM + BV - 1) // BV


def _tp_body(wt_ref, wb_ref, out_ref):
    t = wt_ref[...].T
    out_ref[:, 0:D] = t
    out_ref[:, D:D + 1] = wb_ref[...]
    out_ref[:, D + 1:DP] = jnp.zeros((BV, DP - D - 1), jnp.float32)


def _build_table(wt, wb):
    return pl.pallas_call(
        _tp_body,
        grid=(NVB,),
        in_specs=[
            pl.BlockSpec((D, BV), lambda i: (0, i)),
            pl.BlockSpec((BV, 1), lambda i: (i, 0)),
        ],
        out_specs=pl.BlockSpec((BV, DP), lambda i: (i, 0)),
        out_shape=jax.ShapeDtypeStruct((NVB * BV, DP), jnp.float32),
    )(wt, wb)


def _fm_body(x_hbm, wf_hbm, out_hbm, idx_v, rows_v, out_v, sf0, sf1):
    cid = lax.axis_index("c")
    sid = lax.axis_index("s")
    wid = sid * 2 + cid

    pltpu.sync_copy(x_hbm.at[pl.ds(wid * IPW, IPW)], idx_v)

    sf = (sf0, sf1)

    def start(blk, buf):
        pltpu.make_async_copy(
            wf_hbm.at[idx_v.at[pl.ds(blk * RPB, RPB)]], rows_v.at[buf],
            sf[buf]).start()

    def wait(blk, buf):
        pltpu.make_async_copy(
            wf_hbm.at[idx_v.at[pl.ds(blk * RPB, RPB)]], rows_v.at[buf],
            sf[buf]).wait()

    start(0, 0)
    start(1, 1)

    lane = lax.iota(jnp.int32, 16)
    zeros16 = jnp.zeros((16,), jnp.float32)
    perms = [lane ^ (1 << k) for k in range(4)]

    def lane_sum(v):
        for p in perms:
            v = v + v.at[p].get(mode="promise_in_bounds", unique_indices=True)
        return v

    def group_body(g, _):
        out_vec = zeros16
        for t in range(GRP):
            blk = GRP * g + t
            buf = t % 2
            wait(blk, buf)

            def elem_body(e, out_vec):
                base = e * F_NUM
                acc = rows_v[buf, base, pl.ds(D, 16)]
                for i in range(1, F_NUM):
                    acc = acc + rows_v[buf, base + i, pl.ds(D, 16)]
                for i in range(F_NUM):
                    for j in range(i + 1, F_NUM):
                        v1 = rows_v[buf, base + i, pl.ds(j * 16, 16)]
                        v2 = rows_v[buf, base + j, pl.ds(i * 16, 16)]
                        acc = acc + v1 * v2
                r = lane_sum(acc)
                return jnp.where(lane == (GRP * t + e), r, out_vec)

            out_vec = lax.fori_loop(0, NB, elem_body, out_vec)
            @pl.when(blk + 2 < NBLK)
            def _():
                start(blk + 2, buf)
        out_v[pl.ds(g * 16, 16)] = out_vec
        return 0

    lax.fori_loop(0, NGRP, group_body, 0)
    pltpu.sync_copy(out_v, out_hbm.at[pl.ds(wid * EPW, EPW)])


@jax.jit
def kernel(x, W_bias, W_factor):
    x1 = x.astype(jnp.int32).reshape(-1)
    wt = W_factor.reshape(V_NUM, D).T
    wfp = _build_table(wt, W_bias)

    fm = pl.kernel(
        _fm_body,
        out_type=jax.ShapeDtypeStruct((BATCH,), jnp.float32),
        mesh=plsc.VectorSubcoreMesh(core_axis_name="c", subcore_axis_name="s"),
        scratch_types=[
            pltpu.VMEM((IPW,), jnp.int32),
            pltpu.VMEM((2, RPB, DP), jnp.float32),
            pltpu.VMEM((EPW,), jnp.float32),
            pltpu.SemaphoreType.DMA,
            pltpu.SemaphoreType.DMA,
        ],
    )
    out = fm(x1, wfp)
    return out.reshape(BATCH, 1)
